# baseline (speedup 1.0000x reference)
import jax
import jax.numpy as jnp
from jax import lax
from jax.experimental import pallas as pl
from jax.experimental.pallas import tpu as pltpu
from jax.experimental.pallas import tpu_sc as plsc

N = 10000
D = 256
Q = 64
E = 160000
EB = 128
NB = 1280
EPAD = NB * EB
NT = 16
NBT = NB // NT
NP = 10240
SLAB = NP // NT

f32 = jnp.float32
i32 = jnp.int32


def _make_msgpass():
  mesh = plsc.VectorSubcoreMesh(core_axis_name="c", subcore_axis_name="s")
  out_type = tuple(
      jax.ShapeDtypeStruct((NP, Q), f32) for _ in range(4))
  NR = 5
  scratch = (
      pltpu.VMEM((NBT, EB), i32),
      pltpu.VMEM((NBT, EB), i32),
      pltpu.VMEM((NR, EB, Q), f32),
      pltpu.VMEM((EB, Q), f32),
      pltpu.VMEM_SHARED((NP, Q), f32),
      pltpu.SemaphoreType.DMA((NR,)),
      pltpu.SemaphoreType.DMA((NR,)),
  )

  def body(xq0, xq1, xq2, xq3, src2, dst2, a0, a1, a2, a3,
           src_v, dst_v, rows_v, zbuf, accum, gsem, ssem):
    c = lax.axis_index("c")
    s = lax.axis_index("s")
    z16 = jnp.zeros((16,), f32)

    def zrow(r, carry):
      for j in range(Q // 16):
        zbuf[r, pl.ds(j * 16, 16)] = z16
      return carry
    lax.fori_loop(0, EB, zrow, 0)
    pltpu.sync_copy(src2.at[pl.ds(s * NBT, NBT)], src_v)
    pltpu.sync_copy(dst2.at[pl.ds(s * NBT, NBT)], dst_v)

    def one_pass(x_ref, agg_ref):
      for k in range(SLAB // EB):
        pltpu.sync_copy(zbuf, accum.at[pl.ds(s * SLAB + k * EB, EB)])
      plsc.subcore_barrier()

      def gissue(t, b):
        pltpu.async_copy(x_ref.at[src_v.at[t]], rows_v.at[b], gsem.at[b])

      def gwait(t, b):
        pltpu.make_async_copy(x_ref.at[src_v.at[t]], rows_v.at[b],
                              gsem.at[b]).wait()

      def sissue(t, b):
        pltpu.async_copy(rows_v.at[b], accum.at[dst_v.at[t]], ssem.at[b],
                         add=True)

      def swait(t, b):
        pltpu.make_async_copy(rows_v.at[b], accum.at[dst_v.at[t]],
                              ssem.at[b]).wait()

      for b in range(NR):
        gissue(b, b)

      def quad(t4, carry):
        for b in range(NR):
          t = t4 * NR + b

          @pl.when(jnp.logical_and(t >= 1, t + NR <= NBT))
          def _():
            swait(t - 1, (b - 1) % NR)
            gissue(t + NR - 1, (b + NR - 1) % NR)

          gwait(t, b)
          sissue(t, b)
        return carry
      lax.fori_loop(0, NBT // NR, quad, 0)
      for k in range(NR):
        swait(NBT - NR + k, k)
      plsc.subcore_barrier()
      pltpu.sync_copy(accum.at[pl.ds(s * SLAB, SLAB)],
                      agg_ref.at[pl.ds(s * SLAB, SLAB)])
      plsc.subcore_barrier()

    @pl.when(c == 0)
    def _():
      one_pass(xq0, a0)
      one_pass(xq1, a1)

    @pl.when(c == 1)
    def _():
      one_pass(xq2, a2)
      one_pass(xq3, a3)

  return pl.kernel(body, out_type=out_type, mesh=mesh,
                   scratch_types=scratch,
                   compiler_params=pltpu.CompilerParams(
                       use_tc_tiling_on_sc=False))


DN = NP // 2
DNP = DN + EB
DSLAB = DNP // NT


def _make_deg():
  mesh = plsc.VectorSubcoreMesh(core_axis_name="c", subcore_axis_name="s")
  out_type = (
      jax.ShapeDtypeStruct((DNP, 16), f32),
      jax.ShapeDtypeStruct((DNP, 16), f32),
  )
  scratch = (
      pltpu.VMEM((NBT, EB), i32),
      pltpu.VMEM((EB, 16), f32),
      pltpu.VMEM_SHARED((DNP, 16), f32),
      pltpu.SemaphoreType.DMA,
  )

  def body(dst2, deg0, deg1, dst_v, ones_v, deg_sp, dsem):
    c = lax.axis_index("c")
    s = lax.axis_index("s")
    z16 = jnp.zeros((16,), f32)
    ones16 = jnp.ones((16,), f32)

    def zrow(r, carry):
      ones_v[r, :] = z16
      return carry
    lax.fori_loop(0, EB, zrow, 0)
    for k in range(2):
      pltpu.sync_copy(ones_v, deg_sp.at[pl.ds(s * DSLAB + k * EB, EB)])
    pltpu.sync_copy(ones_v.at[pl.ds(0, DSLAB - 2 * EB)],
                    deg_sp.at[pl.ds(s * DSLAB + 2 * EB, DSLAB - 2 * EB)])

    def orow(r, carry):
      ones_v[r, :] = ones16
      return carry
    lax.fori_loop(0, EB, orow, 0)
    pltpu.sync_copy(dst2.at[pl.ds(s * NBT, NBT)], dst_v)
    lo = c * DN

    def shift(t, carry):
      for j in range(8):
        v = dst_v[t, pl.ds(j * 16, 16)] - lo
        oob = jnp.logical_or(v < 0, v >= DN)
        dst_v[t, pl.ds(j * 16, 16)] = jnp.where(oob, DN, v)
      return carry
    lax.fori_loop(0, NBT, shift, 0)
    plsc.subcore_barrier()

    CH = 16

    def chunk(ci, carry):
      def fire(i, carry2):
        pltpu.async_copy(ones_v, deg_sp.at[dst_v.at[ci * CH + i]], dsem,
                         add=True)
        return carry2
      lax.fori_loop(0, CH, fire, 0)

      def drain(i, carry2):
        pltpu.make_async_copy(ones_v, deg_sp.at[dst_v.at[ci * CH + i]],
                              dsem).wait()
        return carry2
      lax.fori_loop(0, CH, drain, 0)
      return carry
    lax.fori_loop(0, NBT // CH, chunk, 0)
    plsc.subcore_barrier()

    @pl.when(c == 0)
    def _():
      pltpu.sync_copy(deg_sp.at[pl.ds(s * DSLAB, DSLAB)],
                      deg0.at[pl.ds(s * DSLAB, DSLAB)])

    @pl.when(c == 1)
    def _():
      pltpu.sync_copy(deg_sp.at[pl.ds(s * DSLAB, DSLAB)],
                      deg1.at[pl.ds(s * DSLAB, DSLAB)])

  return pl.kernel(body, out_type=out_type, mesh=mesh,
                   scratch_types=scratch)


_msgpass = _make_msgpass()
_deg_call = _make_deg()


BR = 2000
GR = N // BR


def _ln(x, g, b):
  m = jnp.mean(x, axis=1, keepdims=True)
  xc = x - m
  v = jnp.mean(xc * xc, axis=1, keepdims=True)
  return xc * lax.rsqrt(v + 1e-5) * g + b


def _dot(a, b):
  return lax.dot_general(a, b, (((1,), (0,)), ((), ())),
                         preferred_element_type=f32)


def _mix_body(struct, svd, gs, sw, sb, lsg, lsb, lvg, lvb, mix, gw, gb, out):
  sp = _dot(struct[...], sw[...]) + sb[...]
  sn = _ln(sp, lsg[...], lsb[...])
  vn = _ln(svd[...], lvg[...], lvb[...])
  a = jax.nn.sigmoid(mix[...])
  gctx = _dot(gs[...], gw[...]) + gb[...]
  out[...] = a * sn + (1.0 - a) * vn + gctx


def _full(shape):
  return pl.BlockSpec(shape, lambda i: (0, 0))


_mix_call = pl.pallas_call(
    _mix_body,
    grid=(GR,),
    in_specs=[
        pl.BlockSpec((BR, 8), lambda i: (i, 0)),
        pl.BlockSpec((BR, D), lambda i: (i, 0)),
        _full((1, 8)), _full((8, D)), _full((1, D)),
        _full((1, D)), _full((1, D)), _full((1, D)), _full((1, D)),
        _full((1, 1)), _full((8, D)), _full((1, D)),
    ],
    out_specs=pl.BlockSpec((BR, D), lambda i: (i, 0)),
    out_shape=jax.ShapeDtypeStruct((N, D), f32),
)


def _rc_body(deg0, deg1, out):
  dcol = jnp.concatenate(
      [deg0[pl.ds(0, DN), pl.ds(0, 1)],
       deg1[pl.ds(0, N - DN), pl.ds(0, 1)]], axis=0)
  out[...] = 1.0 / jnp.maximum(dcol, 1.0)


_rc_call = pl.pallas_call(
    _rc_body,
    out_shape=jax.ShapeDtypeStruct((N, 1), f32),
)


def _mm_body(a0, a1, a2, a3, rc, x0, x1, x2, x3,
             wl0, wl1, wl2, wl3, wr0, wr1, wr2, wr3, bl, out):
  aggs = (a0, a1, a2, a3)
  xs = (x0, x1, x2, x3)
  wls = (wl0, wl1, wl2, wl3)
  wrs = (wr0, wr1, wr2, wr3)
  r = rc[...]
  y = bl[...]
  for q in range(4):
    y = y + _dot(aggs[q][...] * r, wls[q][...])
    y = y + _dot(xs[q][...], wrs[q][...])
  out[...] = y


_mm_call = pl.pallas_call(
    _mm_body,
    grid=(GR,),
    in_specs=(
        [pl.BlockSpec((BR, Q), lambda i: (i, 0)) for _ in range(4)]
        + [pl.BlockSpec((BR, 1), lambda i: (i, 0))]
        + [pl.BlockSpec((BR, Q), lambda i: (i, 0)) for _ in range(4)]
        + [_full((Q, D)) for _ in range(8)]
        + [_full((1, D))]
    ),
    out_specs=pl.BlockSpec((BR, D), lambda i: (i, 0)),
    out_shape=jax.ShapeDtypeStruct((N, D), f32),
)


def _make_bn(last):
  def body(y, bg, bb, *outs):
    x = y[...]
    m = jnp.mean(x, axis=0, keepdims=True)
    xc = x - m
    v = jnp.mean(xc * xc, axis=0, keepdims=True)
    z = xc * lax.rsqrt(v + 1e-5) * bg[...] + bb[...]
    if last:
      outs[0][...] = z
    else:
      z = jnp.maximum(z, 0.0)
      for q in range(4):
        outs[q][...] = z[:, q * Q:(q + 1) * Q]

  if last:
    out_shape = (jax.ShapeDtypeStruct((N, D), f32),)
  else:
    out_shape = tuple(
        jax.ShapeDtypeStruct((N, Q), f32) for _ in range(4))
  return pl.pallas_call(body, out_shape=out_shape)


_bn_mid = _make_bn(False)
_bn_last = _make_bn(True)


def kernel(struct, svd, edge_index, graph_summary, params):
  src = edge_index[0]
  dst = edge_index[1]
  src2 = jnp.concatenate(
      [src, jnp.zeros((EPAD - E,), i32)]).reshape(NB, EB)
  dst2 = jnp.concatenate(
      [dst, jnp.full((EPAD - E,), N, i32)]).reshape(NB, EB)

  sw = jnp.pad(params['struct_w'], ((0, 2), (0, 0)))
  structp = jnp.pad(struct, ((0, 0), (0, 2)))
  gs = jnp.pad(graph_summary, (0, 5)).reshape(1, 8)
  gw = jnp.pad(params['graph_w'], ((0, 5), (0, 0)))

  def row(v):
    return v.reshape(1, -1)

  xpre = _mix_call(structp, svd, gs, sw, row(params['struct_b']),
                   row(params['ln_s_g']), row(params['ln_s_b']),
                   row(params['ln_v_g']), row(params['ln_v_b']),
                   params['mix'].reshape(1, 1), gw, row(params['graph_b']))
  xq = _bn_mid(xpre, row(params['bn0_g']), row(params['bn0_b']))

  deg0, deg1 = _deg_call(dst2)
  rc = _rc_call(deg0, deg1)
  for i in range(3):
    aggs = _msgpass(*xq, src2, dst2)
    wl = params['l%d_wl' % i]
    wr = params['l%d_wr' % i]
    wqs = [wl[q * Q:(q + 1) * Q] for q in range(4)]
    wrs = [wr[q * Q:(q + 1) * Q] for q in range(4)]
    y = _mm_call(*aggs, rc, *xq, *wqs, *wrs, row(params['l%d_bl' % i]))
    if i < 2:
      xq = _bn_mid(y, row(params['l%d_bn_g' % i]), row(params['l%d_bn_b' % i]))
    else:
      out, = _bn_last(y, row(params['l%d_bn_g' % i]), row(params['l%d_bn_b' % i]))
  return out


# speedup vs baseline: 1.0031x; 1.0031x over previous
import jax
import jax.numpy as jnp
from jax import lax
from jax.experimental import pallas as pl
from jax.experimental.pallas import tpu as pltpu
from jax.experimental.pallas import tpu_sc as plsc

N = 10000
D = 256
Q = 64
E = 160000
EB = 128
NB = 1280
EPAD = NB * EB
NT = 16
NBT = NB // NT
NP = 10240
SLAB = NP // NT

f32 = jnp.float32
i32 = jnp.int32


def _make_msgpass():
  mesh = plsc.VectorSubcoreMesh(core_axis_name="c", subcore_axis_name="s")
  out_type = tuple(
      jax.ShapeDtypeStruct((NP, Q), f32) for _ in range(4))
  NR = 4
  scratch = (
      pltpu.VMEM((NBT, EB), i32),
      pltpu.VMEM((NBT, EB), i32),
      pltpu.VMEM((NR, EB, Q), f32),
      pltpu.VMEM((EB, Q), f32),
      pltpu.VMEM_SHARED((NP, Q), f32),
      pltpu.SemaphoreType.DMA((NR,)),
      pltpu.SemaphoreType.DMA((NR,)),
  )

  def body(xq0, xq1, xq2, xq3, src2, dst2, a0, a1, a2, a3,
           src_v, dst_v, rows_v, zbuf, accum, gsem, ssem):
    c = lax.axis_index("c")
    s = lax.axis_index("s")
    z16 = jnp.zeros((16,), f32)

    def zrow(r, carry):
      for j in range(Q // 16):
        zbuf[r, pl.ds(j * 16, 16)] = z16
      return carry
    lax.fori_loop(0, EB, zrow, 0)
    pltpu.sync_copy(src2.at[pl.ds(s * NBT, NBT)], src_v)
    pltpu.sync_copy(dst2.at[pl.ds(s * NBT, NBT)], dst_v)

    def one_pass(x_ref, agg_ref):
      for k in range(SLAB // EB):
        pltpu.sync_copy(zbuf, accum.at[pl.ds(s * SLAB + k * EB, EB)])
      plsc.subcore_barrier()

      def gissue(t, b):
        pltpu.async_copy(x_ref.at[src_v.at[t]], rows_v.at[b], gsem.at[b])

      def gwait(t, b):
        pltpu.make_async_copy(x_ref.at[src_v.at[t]], rows_v.at[b],
                              gsem.at[b]).wait()

      def sissue(t, b):
        pltpu.async_copy(rows_v.at[b], accum.at[dst_v.at[t]], ssem.at[b],
                         add=True)

      def swait(t, b):
        pltpu.make_async_copy(rows_v.at[b], accum.at[dst_v.at[t]],
                              ssem.at[b]).wait()

      for b in range(NR):
        gissue(b, b)

      def quad(t4, carry):
        for b in range(NR):
          t = t4 * NR + b

          @pl.when(jnp.logical_and(t >= 1, t + NR <= NBT))
          def _():
            swait(t - 1, (b - 1) % NR)
            gissue(t + NR - 1, (b + NR - 1) % NR)

          gwait(t, b)
          sissue(t, b)
        return carry
      lax.fori_loop(0, NBT // NR, quad, 0)
      for k in range(NR):
        swait(NBT - NR + k, k)
      plsc.subcore_barrier()
      pltpu.sync_copy(accum.at[pl.ds(s * SLAB, SLAB)],
                      agg_ref.at[pl.ds(s * SLAB, SLAB)])
      plsc.subcore_barrier()

    @pl.when(c == 0)
    def _():
      one_pass(xq0, a0)
      one_pass(xq1, a1)

    @pl.when(c == 1)
    def _():
      one_pass(xq2, a2)
      one_pass(xq3, a3)

  return pl.kernel(body, out_type=out_type, mesh=mesh,
                   scratch_types=scratch,
                   compiler_params=pltpu.CompilerParams(
                       use_tc_tiling_on_sc=False))


DN = NP // 2
DNP = DN + EB
DSLAB = DNP // NT


def _make_deg():
  mesh = plsc.VectorSubcoreMesh(core_axis_name="c", subcore_axis_name="s")
  out_type = (
      jax.ShapeDtypeStruct((DNP, 16), f32),
      jax.ShapeDtypeStruct((DNP, 16), f32),
  )
  scratch = (
      pltpu.VMEM((NBT, EB), i32),
      pltpu.VMEM((EB, 16), f32),
      pltpu.VMEM_SHARED((DNP, 16), f32),
      pltpu.SemaphoreType.DMA,
  )

  def body(dst2, deg0, deg1, dst_v, ones_v, deg_sp, dsem):
    c = lax.axis_index("c")
    s = lax.axis_index("s")
    z16 = jnp.zeros((16,), f32)
    ones16 = jnp.ones((16,), f32)

    def zrow(r, carry):
      ones_v[r, :] = z16
      return carry
    lax.fori_loop(0, EB, zrow, 0)
    for k in range(2):
      pltpu.sync_copy(ones_v, deg_sp.at[pl.ds(s * DSLAB + k * EB, EB)])
    pltpu.sync_copy(ones_v.at[pl.ds(0, DSLAB - 2 * EB)],
                    deg_sp.at[pl.ds(s * DSLAB + 2 * EB, DSLAB - 2 * EB)])

    def orow(r, carry):
      ones_v[r, :] = ones16
      return carry
    lax.fori_loop(0, EB, orow, 0)
    pltpu.sync_copy(dst2.at[pl.ds(s * NBT, NBT)], dst_v)
    lo = c * DN

    def shift(t, carry):
      for j in range(8):
        v = dst_v[t, pl.ds(j * 16, 16)] - lo
        oob = jnp.logical_or(v < 0, v >= DN)
        dst_v[t, pl.ds(j * 16, 16)] = jnp.where(oob, DN, v)
      return carry
    lax.fori_loop(0, NBT, shift, 0)
    plsc.subcore_barrier()

    CH = 16

    def chunk(ci, carry):
      def fire(i, carry2):
        pltpu.async_copy(ones_v, deg_sp.at[dst_v.at[ci * CH + i]], dsem,
                         add=True)
        return carry2
      lax.fori_loop(0, CH, fire, 0)

      def drain(i, carry2):
        pltpu.make_async_copy(ones_v, deg_sp.at[dst_v.at[ci * CH + i]],
                              dsem).wait()
        return carry2
      lax.fori_loop(0, CH, drain, 0)
      return carry
    lax.fori_loop(0, NBT // CH, chunk, 0)
    plsc.subcore_barrier()

    @pl.when(c == 0)
    def _():
      pltpu.sync_copy(deg_sp.at[pl.ds(s * DSLAB, DSLAB)],
                      deg0.at[pl.ds(s * DSLAB, DSLAB)])

    @pl.when(c == 1)
    def _():
      pltpu.sync_copy(deg_sp.at[pl.ds(s * DSLAB, DSLAB)],
                      deg1.at[pl.ds(s * DSLAB, DSLAB)])

  return pl.kernel(body, out_type=out_type, mesh=mesh,
                   scratch_types=scratch)


_msgpass = _make_msgpass()
_deg_call = _make_deg()


BR = 2000
GR = N // BR


def _ln(x, g, b):
  m = jnp.mean(x, axis=1, keepdims=True)
  xc = x - m
  v = jnp.mean(xc * xc, axis=1, keepdims=True)
  return xc * lax.rsqrt(v + 1e-5) * g + b


def _dot(a, b):
  return lax.dot_general(a, b, (((1,), (0,)), ((), ())),
                         preferred_element_type=f32)


def _mix_body(struct, svd, gs, sw, sb, lsg, lsb, lvg, lvb, mix, gw, gb, out):
  sp = _dot(struct[...], sw[...]) + sb[...]
  sn = _ln(sp, lsg[...], lsb[...])
  vn = _ln(svd[...], lvg[...], lvb[...])
  a = jax.nn.sigmoid(mix[...])
  gctx = _dot(gs[...], gw[...]) + gb[...]
  out[...] = a * sn + (1.0 - a) * vn + gctx


def _full(shape):
  return pl.BlockSpec(shape, lambda i: (0, 0))


_mix_call = pl.pallas_call(
    _mix_body,
    grid=(GR,),
    in_specs=[
        pl.BlockSpec((BR, 8), lambda i: (i, 0)),
        pl.BlockSpec((BR, D), lambda i: (i, 0)),
        _full((1, 8)), _full((8, D)), _full((1, D)),
        _full((1, D)), _full((1, D)), _full((1, D)), _full((1, D)),
        _full((1, 1)), _full((8, D)), _full((1, D)),
    ],
    out_specs=pl.BlockSpec((BR, D), lambda i: (i, 0)),
    out_shape=jax.ShapeDtypeStruct((N, D), f32),
)


def _rc_body(deg0, deg1, out):
  dcol = jnp.concatenate(
      [deg0[pl.ds(0, DN), pl.ds(0, 1)],
       deg1[pl.ds(0, N - DN), pl.ds(0, 1)]], axis=0)
  out[...] = 1.0 / jnp.maximum(dcol, 1.0)


_rc_call = pl.pallas_call(
    _rc_body,
    out_shape=jax.ShapeDtypeStruct((N, 1), f32),
)


def _mm_body(a0, a1, a2, a3, rc, x0, x1, x2, x3,
             wl0, wl1, wl2, wl3, wr0, wr1, wr2, wr3, bl, out):
  aggs = (a0, a1, a2, a3)
  xs = (x0, x1, x2, x3)
  wls = (wl0, wl1, wl2, wl3)
  wrs = (wr0, wr1, wr2, wr3)
  r = rc[...]
  y = bl[...]
  for q in range(4):
    y = y + _dot(aggs[q][...] * r, wls[q][...])
    y = y + _dot(xs[q][...], wrs[q][...])
  out[...] = y


_mm_call = pl.pallas_call(
    _mm_body,
    grid=(GR,),
    in_specs=(
        [pl.BlockSpec((BR, Q), lambda i: (i, 0)) for _ in range(4)]
        + [pl.BlockSpec((BR, 1), lambda i: (i, 0))]
        + [pl.BlockSpec((BR, Q), lambda i: (i, 0)) for _ in range(4)]
        + [_full((Q, D)) for _ in range(8)]
        + [_full((1, D))]
    ),
    out_specs=pl.BlockSpec((BR, D), lambda i: (i, 0)),
    out_shape=jax.ShapeDtypeStruct((N, D), f32),
)


def _make_bn(last):
  def body(y, bg, bb, *outs):
    x = y[...]
    m = jnp.mean(x, axis=0, keepdims=True)
    xc = x - m
    v = jnp.mean(xc * xc, axis=0, keepdims=True)
    z = xc * lax.rsqrt(v + 1e-5) * bg[...] + bb[...]
    if last:
      outs[0][...] = z
    else:
      z = jnp.maximum(z, 0.0)
      for q in range(4):
        outs[q][...] = z[:, q * Q:(q + 1) * Q]

  if last:
    out_shape = (jax.ShapeDtypeStruct((N, D), f32),)
  else:
    out_shape = tuple(
        jax.ShapeDtypeStruct((N, Q), f32) for _ in range(4))
  return pl.pallas_call(body, out_shape=out_shape)


_bn_mid = _make_bn(False)
_bn_last = _make_bn(True)


def kernel(struct, svd, edge_index, graph_summary, params):
  src = edge_index[0]
  dst = edge_index[1]
  src2 = jnp.concatenate(
      [src, jnp.zeros((EPAD - E,), i32)]).reshape(NB, EB)
  dst2 = jnp.concatenate(
      [dst, jnp.full((EPAD - E,), N, i32)]).reshape(NB, EB)

  sw = jnp.pad(params['struct_w'], ((0, 2), (0, 0)))
  structp = jnp.pad(struct, ((0, 0), (0, 2)))
  gs = jnp.pad(graph_summary, (0, 5)).reshape(1, 8)
  gw = jnp.pad(params['graph_w'], ((0, 5), (0, 0)))

  def row(v):
    return v.reshape(1, -1)

  xpre = _mix_call(structp, svd, gs, sw, row(params['struct_b']),
                   row(params['ln_s_g']), row(params['ln_s_b']),
                   row(params['ln_v_g']), row(params['ln_v_b']),
                   params['mix'].reshape(1, 1), gw, row(params['graph_b']))
  xq = _bn_mid(xpre, row(params['bn0_g']), row(params['bn0_b']))

  deg0, deg1 = _deg_call(dst2)
  rc = _rc_call(deg0, deg1)
  for i in range(3):
    aggs = _msgpass(*xq, src2, dst2)
    wl = params['l%d_wl' % i]
    wr = params['l%d_wr' % i]
    wqs = [wl[q * Q:(q + 1) * Q] for q in range(4)]
    wrs = [wr[q * Q:(q + 1) * Q] for q in range(4)]
    y = _mm_call(*aggs, rc, *xq, *wqs, *wrs, row(params['l%d_bl' % i]))
    if i < 2:
      xq = _bn_mid(y, row(params['l%d_bn_g' % i]), row(params['l%d_bn_b' % i]))
    else:
      out, = _bn_last(y, row(params['l%d_bn_g' % i]), row(params['l%d_bn_b' % i]))
  return out
